# baseline (speedup 1.0000x reference)
import functools
import jax
import jax.numpy as jnp
from jax import lax
from jax.experimental import pallas as pl
from jax.experimental.pallas import tpu as pltpu
from jax.experimental.pallas import tpu_sc as plsc

_B, _S, _D, _U = 64, 8192, 128, 16
_NB = 4
_CH = 248
_NFULL = (_S - _U) // _CH
_TAIL = (_S - _U) - _NFULL * _CH
_ROWS = [_CH] * _NFULL + ([_TAIL] if _TAIL else [])


def _sc_body(cache_hbm, update_hbm, out_hbm, buf, in_sems, out_sems):
    c = lax.axis_index("c")
    s = lax.axis_index("s")
    wid = c * 16 + s

    jobs = []
    for b_i in range(2):
        for k, r in enumerate(_ROWS):
            jobs.append((b_i, 0, k * _CH, r))
        jobs.append((b_i, 1, 0, _U))
    J = len(jobs)

    def src_of(j):
        b_i, kind, off, r = jobs[j]
        b = wid * 2 + b_i
        if kind == 0:
            return cache_hbm.at[b, pl.ds(_U + off, r), :]
        return update_hbm.at[b, :, :]

    def dst_of(j):
        b_i, kind, off, r = jobs[j]
        b = wid * 2 + b_i
        if kind == 0:
            return out_hbm.at[b, pl.ds(off, r), :]
        return out_hbm.at[b, pl.ds(_S - _U, _U), :]

    def bufslice(j):
        r = jobs[j][3]
        return buf.at[s, j % _NB, pl.ds(0, r), :]

    def start_in(j):
        pltpu.make_async_copy(src_of(j), bufslice(j), in_sems.at[j % _NB]).start()

    def wait_in(j):
        pltpu.make_async_copy(src_of(j), bufslice(j), in_sems.at[j % _NB]).wait()

    def start_out(j):
        pltpu.make_async_copy(bufslice(j), dst_of(j), out_sems.at[j % _NB]).start()

    def wait_out(j):
        pltpu.make_async_copy(bufslice(j), dst_of(j), out_sems.at[j % _NB]).wait()

    for j in range(_NB - 1):
        start_in(j)
    for j in range(J):
        if j + _NB - 1 < J:
            if j >= 1:
                wait_out(j - 1)
            start_in(j + _NB - 1)
        wait_in(j)
        start_out(j)
    for j in range(max(0, J - _NB), J):
        wait_out(j)


def kernel(cache, update):
    mesh = plsc.VectorSubcoreMesh(core_axis_name="c", subcore_axis_name="s")
    k = functools.partial(
        pl.kernel,
        mesh=mesh,
        out_type=jax.ShapeDtypeStruct((_B, _S, _D), jnp.float32),
        scratch_types=[
            pltpu.VMEM_SHARED((16, _NB, _CH, _D), jnp.float32),
            pltpu.SemaphoreType.DMA((_NB,)),
            pltpu.SemaphoreType.DMA((_NB,)),
        ],
    )(_sc_body)
    return k(cache, update)


# speedup vs baseline: 1.0095x; 1.0095x over previous
import functools
import jax
import jax.numpy as jnp
from jax import lax
from jax.experimental import pallas as pl
from jax.experimental.pallas import tpu as pltpu
from jax.experimental.pallas import tpu_sc as plsc

_B, _S, _D, _U = 64, 8192, 128, 16
_NB = 3
_CH = 336
_NFULL = (_S - _U) // _CH
_TAIL = (_S - _U) - _NFULL * _CH
_ROWS = [_CH] * _NFULL + ([_TAIL] if _TAIL else [])


def _sc_body(cache_hbm, update_hbm, out_hbm, buf, in_sems, out_sems):
    c = lax.axis_index("c")
    s = lax.axis_index("s")
    wid = c * 16 + s

    jobs = []
    for b_i in range(2):
        for k, r in enumerate(_ROWS):
            jobs.append((b_i, 0, k * _CH, r))
        jobs.append((b_i, 1, 0, _U))
    J = len(jobs)

    def src_of(j):
        b_i, kind, off, r = jobs[j]
        b = wid * 2 + b_i
        if kind == 0:
            return cache_hbm.at[b, pl.ds(_U + off, r), :]
        return update_hbm.at[b, :, :]

    def dst_of(j):
        b_i, kind, off, r = jobs[j]
        b = wid * 2 + b_i
        if kind == 0:
            return out_hbm.at[b, pl.ds(off, r), :]
        return out_hbm.at[b, pl.ds(_S - _U, _U), :]

    def bufslice(j):
        r = jobs[j][3]
        return buf.at[s, j % _NB, pl.ds(0, r), :]

    def start_in(j):
        pltpu.make_async_copy(src_of(j), bufslice(j), in_sems.at[j % _NB]).start()

    def wait_in(j):
        pltpu.make_async_copy(src_of(j), bufslice(j), in_sems.at[j % _NB]).wait()

    def start_out(j):
        pltpu.make_async_copy(bufslice(j), dst_of(j), out_sems.at[j % _NB]).start()

    def wait_out(j):
        pltpu.make_async_copy(bufslice(j), dst_of(j), out_sems.at[j % _NB]).wait()

    for j in range(_NB - 1):
        start_in(j)
    for j in range(J):
        if j + _NB - 1 < J:
            if j >= 1:
                wait_out(j - 1)
            start_in(j + _NB - 1)
        wait_in(j)
        start_out(j)
    for j in range(max(0, J - _NB), J):
        wait_out(j)


def kernel(cache, update):
    mesh = plsc.VectorSubcoreMesh(core_axis_name="c", subcore_axis_name="s")
    k = functools.partial(
        pl.kernel,
        mesh=mesh,
        out_type=jax.ShapeDtypeStruct((_B, _S, _D), jnp.float32),
        scratch_types=[
            pltpu.VMEM_SHARED((16, _NB, _CH, _D), jnp.float32),
            pltpu.SemaphoreType.DMA((_NB,)),
            pltpu.SemaphoreType.DMA((_NB,)),
        ],
    )(_sc_body)
    return k(cache, update)
